# Initial kernel scaffold; baseline (speedup 1.0000x reference)
import jax
import jax.numpy as jnp
from jax.experimental import pallas as pl


def kernel(f_s, f_t, l_t):
    raise NotImplementedError("write your pallas kernel here")


# speedup vs baseline: 5.5687x; 5.5687x over previous
import jax
import jax.numpy as jnp
from jax.experimental import pallas as pl
from jax.experimental.pallas import tpu as pltpu

_N, _C, _H, _W = 128, 1000, 14, 14
_HW = _H * _W
_CAMS = 100
_K = _CAMS - 1
_NT = 8


def _mask_kernel(l_ref, mask_ref):
    x = l_ref[...] + 0.0
    bits = jax.lax.bitcast_convert_type(x, jnp.int32)
    imin = jnp.int32(-(2**31))
    mag = bits & jnp.int32(0x7FFFFFFF)
    key = jnp.where(bits >= 0, bits, -mag)
    u = jnp.zeros((l_ref.shape[0], 1), jnp.int32)
    for b in range(31, -1, -1):
        u_c = u | (imin if b == 31 else jnp.int32(1 << b))
        v_c = u_c ^ imin
        cnt = jnp.sum((key >= v_c).astype(jnp.int32), axis=1, keepdims=True)
        u = jnp.where(cnt >= _CAMS, u_c, u)
    v = u ^ imin
    mask_ref[...] = (key > v).astype(jnp.float32)


def _loss_kernel(fs_ref, ft_ref, w_ref, out_ref):
    i = pl.program_id(0)
    d = (fs_ref[...] - ft_ref[...]).reshape(_NT * _C, _HW)
    l_idx = jax.lax.broadcasted_iota(jnp.int32, (_HW, 4), 0)
    q_of_l = (l_idx // _W // 7) * 2 + ((l_idx % _W) // 7)
    q_idx = jax.lax.broadcasted_iota(jnp.int32, (_HW, 4), 1)
    m = (q_of_l == q_idx).astype(jnp.bfloat16)
    s = jnp.dot(d.astype(jnp.bfloat16), m, preferred_element_type=jnp.float32)
    part = jnp.sum(s * s * w_ref[...])

    @pl.when(i == 0)
    def _():
        out_ref[0, 0] = 0.0

    out_ref[0, 0] += part


def kernel(f_s, f_t, l_t):
    mask = pl.pallas_call(
        _mask_kernel,
        out_shape=jax.ShapeDtypeStruct((_N, _C), jnp.float32),
    )(l_t)
    w = mask.reshape(_N * _C, 1)
    fs3 = f_s.reshape(_N, _C, _HW)
    ft3 = f_t.reshape(_N, _C, _HW)
    grid = _N // _NT
    acc = pl.pallas_call(
        _loss_kernel,
        grid=(grid,),
        in_specs=[
            pl.BlockSpec((_NT, _C, _HW), lambda i: (i, 0, 0)),
            pl.BlockSpec((_NT, _C, _HW), lambda i: (i, 0, 0)),
            pl.BlockSpec((_NT * _C, 1), lambda i: (i, 0)),
        ],
        out_specs=pl.BlockSpec(
            (1, 1), lambda i: (0, 0), memory_space=pltpu.SMEM
        ),
        out_shape=jax.ShapeDtypeStruct((1, 1), jnp.float32),
    )(fs3, ft3, w)
    scale = 1.0 / (49.0 * 49.0 * _N * _K * 4.0)
    return (acc[0, 0] * scale).astype(jnp.float32)


# speedup vs baseline: 30.5938x; 5.4939x over previous
import jax
import jax.numpy as jnp
from jax.experimental import pallas as pl
from jax.experimental.pallas import tpu as pltpu

_N, _C, _H, _W = 128, 1000, 14, 14
_CAMS = 100
_K = _CAMS - 1
_CT = 40


def _mask_kernel(l_ref, mask_ref):
    x = l_ref[...] + 0.0
    bits = jax.lax.bitcast_convert_type(x, jnp.int32)
    imin = jnp.int32(-(2**31))
    mag = bits & jnp.int32(0x7FFFFFFF)
    key = jnp.where(bits >= 0, bits, -mag)
    u = jnp.zeros((1, _N), jnp.int32)
    for b in range(31, -1, -1):
        u_c = u | (imin if b == 31 else jnp.int32(1 << b))
        v_c = u_c ^ imin
        cnt = jnp.sum((key >= v_c).astype(jnp.int32), axis=0, keepdims=True)
        u = jnp.where(cnt >= _CAMS, u_c, u)
    v = u ^ imin
    mask_ref[...] = (key > v).astype(jnp.float32)


def _loss_kernel(fs_ref, ft_ref, w_ref, out_ref):
    i = pl.program_id(0)
    d = fs_ref[...] - ft_ref[...]
    q00 = jnp.sum(d[0:7, 0:7], axis=(0, 1))
    q01 = jnp.sum(d[0:7, 7:14], axis=(0, 1))
    q10 = jnp.sum(d[7:14, 0:7], axis=(0, 1))
    q11 = jnp.sum(d[7:14, 7:14], axis=(0, 1))
    e = q00 * q00 + q01 * q01 + q10 * q10 + q11 * q11
    part = jnp.sum(e * w_ref[...])

    @pl.when(i == 0)
    def _():
        out_ref[0, 0] = 0.0

    out_ref[0, 0] += part


def kernel(f_s, f_t, l_t):
    fs_t = jnp.transpose(f_s, (2, 3, 1, 0))
    ft_t = jnp.transpose(f_t, (2, 3, 1, 0))
    lt_t = jnp.transpose(l_t, (1, 0))
    mask = pl.pallas_call(
        _mask_kernel,
        out_shape=jax.ShapeDtypeStruct((_C, _N), jnp.float32),
    )(lt_t)
    grid = _C // _CT
    acc = pl.pallas_call(
        _loss_kernel,
        grid=(grid,),
        in_specs=[
            pl.BlockSpec((_H, _W, _CT, _N), lambda i: (0, 0, i, 0)),
            pl.BlockSpec((_H, _W, _CT, _N), lambda i: (0, 0, i, 0)),
            pl.BlockSpec((_CT, _N), lambda i: (i, 0)),
        ],
        out_specs=pl.BlockSpec(
            (1, 1), lambda i: (0, 0), memory_space=pltpu.SMEM
        ),
        out_shape=jax.ShapeDtypeStruct((1, 1), jnp.float32),
    )(fs_t, ft_t, mask)
    scale = 1.0 / (49.0 * 49.0 * _N * _K * 4.0)
    return (acc[0, 0] * scale).astype(jnp.float32)


# speedup vs baseline: 31.6010x; 1.0329x over previous
import jax
import jax.numpy as jnp
from jax.experimental import pallas as pl
from jax.experimental.pallas import tpu as pltpu

_N, _C, _H, _W = 128, 1000, 14, 14
_CAMS = 100
_K = _CAMS - 1
_CT = 200


def _loss_kernel(lt_ref, fs_ref, ft_ref, out_ref, mask_ref):
    qh = pl.program_id(0)
    qw = pl.program_id(1)
    ci = pl.program_id(2)

    @pl.when((qh == 0) & (qw == 0) & (ci == 0))
    def _():
        x = lt_ref[...] + 0.0
        bits = jax.lax.bitcast_convert_type(x, jnp.int32)
        imin = jnp.int32(-(2**31))
        mag = bits & jnp.int32(0x7FFFFFFF)
        key = jnp.where(bits >= 0, bits, -mag)
        u = jnp.zeros((1, _N), jnp.int32)
        for b in range(31, -1, -1):
            u_c = u | (imin if b == 31 else jnp.int32(1 << b))
            v_c = u_c ^ imin
            cnt = jnp.sum((key >= v_c).astype(jnp.int32), axis=0, keepdims=True)
            u = jnp.where(cnt >= _CAMS, u_c, u)
        v = u ^ imin
        mask_ref[...] = (key > v).astype(jnp.float32)
        out_ref[0, 0] = 0.0

    d = fs_ref[...] - ft_ref[...]
    s = jnp.sum(d, axis=(0, 1))
    w = mask_ref[pl.ds(ci * _CT, _CT), :]
    out_ref[0, 0] += jnp.sum(s * s * w)


def kernel(f_s, f_t, l_t):
    fs_t = jnp.transpose(f_s, (2, 3, 1, 0))
    ft_t = jnp.transpose(f_t, (2, 3, 1, 0))
    lt_t = jnp.transpose(l_t, (1, 0))
    acc = pl.pallas_call(
        _loss_kernel,
        grid=(2, 2, _C // _CT),
        in_specs=[
            pl.BlockSpec((_C, _N), lambda qh, qw, ci: (0, 0)),
            pl.BlockSpec((7, 7, _CT, _N), lambda qh, qw, ci: (qh, qw, ci, 0)),
            pl.BlockSpec((7, 7, _CT, _N), lambda qh, qw, ci: (qh, qw, ci, 0)),
        ],
        out_specs=pl.BlockSpec(
            (1, 1), lambda qh, qw, ci: (0, 0), memory_space=pltpu.SMEM
        ),
        out_shape=jax.ShapeDtypeStruct((1, 1), jnp.float32),
        scratch_shapes=[pltpu.VMEM((_C, _N), jnp.float32)],
    )(lt_t, fs_t, ft_t)
    scale = 1.0 / (49.0 * 49.0 * _N * _K * 4.0)
    return (acc[0, 0] * scale).astype(jnp.float32)
